# baseline (speedup 1.0000x reference)
import functools

import jax
import jax.numpy as jnp
from jax import lax
from jax.experimental import pallas as pl
from jax.experimental.pallas import tpu as pltpu
from jax.experimental.pallas import tpu_sc as plsc

NUM_EMB = 1_000_000
DIM = 16
BATCH = 16384
HIST = 50

NC = 2
NS = 16
NW = NC * NS
BPW = BATCH // NW
BCH = 128
NCH = BPW // BCH
NCLUS = NUM_EMB // 8

TBLK = 128
NTB = NCLUS // TBLK
REM_CLUS = NTB * TBLK
REM_I = REM_CLUS * 8
TAIL_CLUS = NCLUS - 8


def _transpose_body(tab_hbm, tail_hbm, mu_hbm, in0, in1, out0, out1,
                    isem0, isem1, osem0, osem1):
    wid = lax.axis_index("s") * NC + lax.axis_index("c")
    nblk = (NTB + NW - 1 - wid) // NW
    lane = lax.iota(jnp.int32, 16)

    def blk_i0(k):
        return pl.multiple_of((wid + NW * k) * (TBLK * 8), 1024)

    def fire_block(i0, buf, sem, nbands):
        for p in range(nbands):
            pltpu.async_copy(
                tab_hbm.at[pl.ds(0, DIM), pl.ds(i0 + 128 * p, 128)],
                buf.at[pl.ds(16 * p, DIM)], sem,
            )

    def wait_block(i0, buf, sem, nbands):
        for p in range(nbands):
            pltpu.make_async_copy(
                tab_hbm.at[pl.ds(0, DIM), pl.ds(i0 + 128 * p, 128)],
                buf.at[pl.ds(16 * p, DIM)], sem,
            ).wait()

    def load_block(i0, buf, sem, nbands):
        fire_block(i0, buf, sem, nbands)
        wait_block(i0, buf, sem, nbands)

    rowvs = [lane + 16 * p for p in range(8)]

    def extract(buf, out_v, nbands):
        def row(qq, carry):
            for rr in range(8):
                colv = jnp.full((16,), 8 * qq + rr, jnp.int32)
                for p in range(nbands):
                    out_v[16 * p + qq, pl.ds(rr * DIM, DIM)] = (
                        plsc.load_gather(buf, [rowvs[p], colv])
                    )
            return carry

        lax.fori_loop(0, 16, row, 0)

    def fire_out(k, out_v, sem):
        start = pl.multiple_of((wid + NW * k) * TBLK, TBLK)
        pltpu.async_copy(out_v, mu_hbm.at[pl.ds(start, TBLK)], sem)

    def wait_out(out_v, sem):
        pltpu.make_async_copy(out_v, mu_hbm.at[pl.ds(0, TBLK)], sem).wait()

    @pl.when(nblk > 0)
    def _():
        fire_block(blk_i0(0), in0, isem0, 8)

    def pair_body(kk, carry):
        k0 = 2 * kk
        k1 = k0 + 1
        c0 = k0 < nblk
        c1 = k1 < nblk
        c2 = (k1 + 1) < nblk

        @pl.when(c1)
        def _():
            fire_block(blk_i0(k1), in1, isem1, 8)

        @pl.when(c0)
        def _():
            wait_block(blk_i0(k0), in0, isem0, 8)

        @pl.when(jnp.logical_and(c0, k0 >= 2))
        def _():
            wait_out(out0, osem0)

        @pl.when(c0)
        def _():
            extract(in0, out0, 8)
            fire_out(k0, out0, osem0)

        @pl.when(c2)
        def _():
            fire_block(blk_i0(k1 + 1), in0, isem0, 8)

        @pl.when(c1)
        def _():
            wait_block(blk_i0(k1), in1, isem1, 8)

        @pl.when(jnp.logical_and(c1, k1 >= 2))
        def _():
            wait_out(out1, osem1)

        @pl.when(c1)
        def _():
            extract(in1, out1, 8)
            fire_out(k1, out1, osem1)

        return carry

    lax.fori_loop(0, (NTB // NW + 2) // 2, pair_body, 0)

    @pl.when(nblk >= 1)
    def _():
        wait_out(out0, osem0)

    @pl.when(nblk >= 2)
    def _():
        wait_out(out1, osem1)

    @pl.when(wid == 0)
    def _():
        load_block(REM_I, in0, isem0, 4)
        extract(in0, out0, 4)
        pltpu.sync_copy(
            out0.at[pl.ds(0, TAIL_CLUS - REM_CLUS)],
            mu_hbm.at[pl.ds(REM_CLUS, TAIL_CLUS - REM_CLUS)],
        )

    @pl.when(wid == 1)
    def _():
        pltpu.sync_copy(tail_hbm, out1.at[pl.ds(0, 8)])
        pltpu.sync_copy(out1.at[pl.ds(0, 8)], mu_hbm.at[pl.ds(TAIL_CLUS, 8)])


_transpose_call = functools.partial(
    pl.kernel,
    mesh=plsc.VectorSubcoreMesh(core_axis_name="c", subcore_axis_name="s"),
    out_type=jax.ShapeDtypeStruct((NCLUS, 128), jnp.float32),
    scratch_types=[
        pltpu.VMEM((TBLK, 128), jnp.float32),
        pltpu.VMEM((TBLK, 128), jnp.float32),
        pltpu.VMEM((TBLK, 128), jnp.float32),
        pltpu.VMEM((TBLK, 128), jnp.float32),
        pltpu.SemaphoreType.DMA,
        pltpu.SemaphoreType.DMA,
        pltpu.SemaphoreType.DMA,
        pltpu.SemaphoreType.DMA,
    ],
    compiler_params=pltpu.CompilerParams(
        use_tc_tiling_on_sc=True, needs_layout_passes=False
    ),
)(_transpose_body)


def _gather_body(idx_hbm, mu_hbm, out_hbm, idx_v, q_v, clus0, clus1,
                 ost0, ost1, gsem0, gsem1, osem0, osem1):
    wid = lax.axis_index("s") * NC + lax.axis_index("c")
    b0w = wid * BPW
    lane = lax.iota(jnp.int32, 16)

    def extract(h, clus, ost):
        for bg in range(8):
            sl = pl.ds(bg * 16, 16)
            colb = (idx_v[h, sl] & 7) * 16
            rowv = lane + (bg * 16)
            for d in range(DIM):
                ost[d, sl] = plsc.load_gather(clus, [rowv, colb + d])

    def chunk_body(c, carry):
        b0 = b0w + c * BCH
        pltpu.sync_copy(idx_hbm.at[:, pl.ds(b0, BCH)], idx_v)

        def qrow(h, carry2):
            for i in range(8):
                sl = pl.ds(i * 16, 16)
                q_v[h, sl] = idx_v[h, sl] >> 3
            return carry2

        lax.fori_loop(0, HIST, qrow, 0)

        pltpu.async_copy(mu_hbm.at[q_v.at[0]], clus0, gsem0)

        def pair_body(k, carry2):
            h = 2 * k
            pltpu.async_copy(mu_hbm.at[q_v.at[h + 1]], clus1, gsem1)
            pltpu.make_async_copy(mu_hbm.at[q_v.at[h]], clus0, gsem0).wait()
            extract(h, clus0, ost0)

            @pl.when(k > 0)
            def _():
                pltpu.make_async_copy(
                    ost0, out_hbm.at[h, :, pl.ds(b0, BCH)], osem0
                ).wait()

            pltpu.async_copy(ost0, out_hbm.at[h, :, pl.ds(b0, BCH)], osem0)

            @pl.when(k < (HIST // 2 - 1))
            def _():
                pltpu.async_copy(mu_hbm.at[q_v.at[h + 2]], clus0, gsem0)

            pltpu.make_async_copy(mu_hbm.at[q_v.at[h + 1]], clus1, gsem1).wait()
            extract(h + 1, clus1, ost1)

            @pl.when(k > 0)
            def _():
                pltpu.make_async_copy(
                    ost1, out_hbm.at[h + 1, :, pl.ds(b0, BCH)], osem1
                ).wait()

            pltpu.async_copy(ost1, out_hbm.at[h + 1, :, pl.ds(b0, BCH)], osem1)
            return carry2

        lax.fori_loop(0, HIST // 2, pair_body, 0)

        pltpu.make_async_copy(ost0, out_hbm.at[0, :, pl.ds(b0, BCH)], osem0).wait()
        pltpu.make_async_copy(ost1, out_hbm.at[1, :, pl.ds(b0, BCH)], osem1).wait()
        return carry

    lax.fori_loop(0, NCH, chunk_body, 0)


_gather_call = functools.partial(
    pl.kernel,
    mesh=plsc.VectorSubcoreMesh(core_axis_name="c", subcore_axis_name="s"),
    out_type=jax.ShapeDtypeStruct((HIST, DIM, BATCH), jnp.float32),
    scratch_types=[
        pltpu.VMEM((HIST, BCH), jnp.int32),
        pltpu.VMEM((HIST, BCH), jnp.int32),
        pltpu.VMEM((BCH, 128), jnp.float32),
        pltpu.VMEM((BCH, 128), jnp.float32),
        pltpu.VMEM((DIM, BCH), jnp.float32),
        pltpu.VMEM((DIM, BCH), jnp.float32),
        pltpu.SemaphoreType.DMA,
        pltpu.SemaphoreType.DMA,
        pltpu.SemaphoreType.DMA,
        pltpu.SemaphoreType.DMA,
    ],
    compiler_params=pltpu.CompilerParams(
        use_tc_tiling_on_sc=True, needs_layout_passes=False
    ),
)(_gather_body)


@jax.jit
def kernel(input, embedding_weight):
    idx_t = input.T.astype(jnp.int32)
    tab_t = embedding_weight.T
    tail = embedding_weight[8 * TAIL_CLUS:, :DIM].reshape(8, 128)
    mu_c = _transpose_call(tab_t, tail)
    out_t = _gather_call(idx_t, mu_c)
    return jnp.transpose(out_t, (2, 0, 1))


# speedup vs baseline: 1.3400x; 1.3400x over previous
import functools

import jax
import jax.numpy as jnp
from jax import lax
from jax.experimental import pallas as pl
from jax.experimental.pallas import tpu as pltpu
from jax.experimental.pallas import tpu_sc as plsc

NUM_EMB = 1_000_000
DIM = 16
BATCH = 16384
HIST = 50

NC = 2
NS = 16
NW = NC * NS
BPW = BATCH // NW
BCH = 128
NCH = BPW // BCH
NCLUS = NUM_EMB // 8

TBLK = 128
NTB = NCLUS // TBLK
REM_CLUS = NTB * TBLK
REM_I = REM_CLUS * 8
TAIL_CLUS = NCLUS - 8


def _transpose_body(tab_hbm, tail_hbm, mu_hbm, in0, in1, out0, out1,
                    isem0, isem1, osem0, osem1):
    wid = lax.axis_index("s") * NC + lax.axis_index("c")
    nblk = (NTB + NW - 1 - wid) // NW
    lane = lax.iota(jnp.int32, 16)

    def blk_i0(k):
        return pl.multiple_of((wid + NW * k) * (TBLK * 8), 1024)

    def fire_block(i0, buf, sem, nbands):
        for p in range(nbands):
            pltpu.async_copy(
                tab_hbm.at[pl.ds(0, DIM), pl.ds(i0 + 128 * p, 128)],
                buf.at[pl.ds(16 * p, DIM)], sem,
            )

    def wait_block(i0, buf, sem, nbands):
        for p in range(nbands):
            pltpu.make_async_copy(
                tab_hbm.at[pl.ds(0, DIM), pl.ds(i0 + 128 * p, 128)],
                buf.at[pl.ds(16 * p, DIM)], sem,
            ).wait()

    def load_block(i0, buf, sem, nbands):
        fire_block(i0, buf, sem, nbands)
        wait_block(i0, buf, sem, nbands)

    rowvs = [lane + 16 * p for p in range(8)]

    def extract(buf, out_v, nbands):
        @plsc.parallel_loop(0, 16, unroll=2)
        def row(qq):
            for rr in range(8):
                colv = jnp.full((16,), 8 * qq + rr, jnp.int32)
                for p in range(nbands):
                    out_v[16 * p + qq, pl.ds(rr * DIM, DIM)] = (
                        plsc.load_gather(buf, [rowvs[p], colv])
                    )

    def fire_out(k, out_v, sem):
        start = pl.multiple_of((wid + NW * k) * TBLK, TBLK)
        pltpu.async_copy(out_v, mu_hbm.at[pl.ds(start, TBLK)], sem)

    def wait_out(out_v, sem):
        pltpu.make_async_copy(out_v, mu_hbm.at[pl.ds(0, TBLK)], sem).wait()

    @pl.when(nblk > 0)
    def _():
        fire_block(blk_i0(0), in0, isem0, 8)

    def pair_body(kk, carry):
        k0 = 2 * kk
        k1 = k0 + 1
        c0 = k0 < nblk
        c1 = k1 < nblk
        c2 = (k1 + 1) < nblk

        @pl.when(c1)
        def _():
            fire_block(blk_i0(k1), in1, isem1, 8)

        @pl.when(c0)
        def _():
            wait_block(blk_i0(k0), in0, isem0, 8)

        @pl.when(jnp.logical_and(c0, k0 >= 2))
        def _():
            wait_out(out0, osem0)

        @pl.when(c0)
        def _():
            extract(in0, out0, 8)
            fire_out(k0, out0, osem0)

        @pl.when(c2)
        def _():
            fire_block(blk_i0(k1 + 1), in0, isem0, 8)

        @pl.when(c1)
        def _():
            wait_block(blk_i0(k1), in1, isem1, 8)

        @pl.when(jnp.logical_and(c1, k1 >= 2))
        def _():
            wait_out(out1, osem1)

        @pl.when(c1)
        def _():
            extract(in1, out1, 8)
            fire_out(k1, out1, osem1)

        return carry

    lax.fori_loop(0, (NTB // NW + 2) // 2, pair_body, 0)

    @pl.when(nblk >= 1)
    def _():
        wait_out(out0, osem0)

    @pl.when(nblk >= 2)
    def _():
        wait_out(out1, osem1)

    @pl.when(wid == 0)
    def _():
        load_block(REM_I, in0, isem0, 4)
        extract(in0, out0, 4)
        pltpu.sync_copy(
            out0.at[pl.ds(0, TAIL_CLUS - REM_CLUS)],
            mu_hbm.at[pl.ds(REM_CLUS, TAIL_CLUS - REM_CLUS)],
        )

    @pl.when(wid == 1)
    def _():
        pltpu.sync_copy(tail_hbm, out1.at[pl.ds(0, 8)])
        pltpu.sync_copy(out1.at[pl.ds(0, 8)], mu_hbm.at[pl.ds(TAIL_CLUS, 8)])


_transpose_call = functools.partial(
    pl.kernel,
    mesh=plsc.VectorSubcoreMesh(core_axis_name="c", subcore_axis_name="s"),
    out_type=jax.ShapeDtypeStruct((NCLUS, 128), jnp.float32),
    scratch_types=[
        pltpu.VMEM((TBLK, 128), jnp.float32),
        pltpu.VMEM((TBLK, 128), jnp.float32),
        pltpu.VMEM((TBLK, 128), jnp.float32),
        pltpu.VMEM((TBLK, 128), jnp.float32),
        pltpu.SemaphoreType.DMA,
        pltpu.SemaphoreType.DMA,
        pltpu.SemaphoreType.DMA,
        pltpu.SemaphoreType.DMA,
    ],
    compiler_params=pltpu.CompilerParams(
        use_tc_tiling_on_sc=True, needs_layout_passes=False
    ),
)(_transpose_body)


def _gather_body(idx_hbm, mu_hbm, out_hbm, idx_v, q_v, clus0, clus1,
                 ost0, ost1, gsem0, gsem1, osem0, osem1):
    wid = lax.axis_index("s") * NC + lax.axis_index("c")
    b0w = wid * BPW
    lane = lax.iota(jnp.int32, 16)

    def extract(h, clus, ost):
        for bg in range(8):
            sl = pl.ds(bg * 16, 16)
            colb = (idx_v[h, sl] & 7) * 16
            rowv = lane + (bg * 16)
            for d in range(DIM):
                ost[d, sl] = plsc.load_gather(clus, [rowv, colb + d])

    def chunk_body(c, carry):
        b0 = b0w + c * BCH
        pltpu.sync_copy(idx_hbm.at[:, pl.ds(b0, BCH)], idx_v)

        def qrow(h, carry2):
            for i in range(8):
                sl = pl.ds(i * 16, 16)
                q_v[h, sl] = idx_v[h, sl] >> 3
            return carry2

        lax.fori_loop(0, HIST, qrow, 0)

        pltpu.async_copy(mu_hbm.at[q_v.at[0]], clus0, gsem0)

        def pair_body(k, carry2):
            h = 2 * k
            pltpu.async_copy(mu_hbm.at[q_v.at[h + 1]], clus1, gsem1)
            pltpu.make_async_copy(mu_hbm.at[q_v.at[h]], clus0, gsem0).wait()
            extract(h, clus0, ost0)

            @pl.when(k > 0)
            def _():
                pltpu.make_async_copy(
                    ost0, out_hbm.at[h, :, pl.ds(b0, BCH)], osem0
                ).wait()

            pltpu.async_copy(ost0, out_hbm.at[h, :, pl.ds(b0, BCH)], osem0)

            @pl.when(k < (HIST // 2 - 1))
            def _():
                pltpu.async_copy(mu_hbm.at[q_v.at[h + 2]], clus0, gsem0)

            pltpu.make_async_copy(mu_hbm.at[q_v.at[h + 1]], clus1, gsem1).wait()
            extract(h + 1, clus1, ost1)

            @pl.when(k > 0)
            def _():
                pltpu.make_async_copy(
                    ost1, out_hbm.at[h + 1, :, pl.ds(b0, BCH)], osem1
                ).wait()

            pltpu.async_copy(ost1, out_hbm.at[h + 1, :, pl.ds(b0, BCH)], osem1)
            return carry2

        lax.fori_loop(0, HIST // 2, pair_body, 0)

        pltpu.make_async_copy(ost0, out_hbm.at[0, :, pl.ds(b0, BCH)], osem0).wait()
        pltpu.make_async_copy(ost1, out_hbm.at[1, :, pl.ds(b0, BCH)], osem1).wait()
        return carry

    lax.fori_loop(0, NCH, chunk_body, 0)


_gather_call = functools.partial(
    pl.kernel,
    mesh=plsc.VectorSubcoreMesh(core_axis_name="c", subcore_axis_name="s"),
    out_type=jax.ShapeDtypeStruct((HIST, DIM, BATCH), jnp.float32),
    scratch_types=[
        pltpu.VMEM((HIST, BCH), jnp.int32),
        pltpu.VMEM((HIST, BCH), jnp.int32),
        pltpu.VMEM((BCH, 128), jnp.float32),
        pltpu.VMEM((BCH, 128), jnp.float32),
        pltpu.VMEM((DIM, BCH), jnp.float32),
        pltpu.VMEM((DIM, BCH), jnp.float32),
        pltpu.SemaphoreType.DMA,
        pltpu.SemaphoreType.DMA,
        pltpu.SemaphoreType.DMA,
        pltpu.SemaphoreType.DMA,
    ],
    compiler_params=pltpu.CompilerParams(
        use_tc_tiling_on_sc=True, needs_layout_passes=False
    ),
)(_gather_body)


@jax.jit
def kernel(input, embedding_weight):
    idx_t = input.T.astype(jnp.int32)
    tab_t = embedding_weight.T
    tail = embedding_weight[8 * TAIL_CLUS:, :DIM].reshape(8, 128)
    mu_c = _transpose_call(tab_t, tail)
    out_t = _gather_call(idx_t, mu_c)
    return jnp.transpose(out_t, (2, 0, 1))
